# baseline (speedup 1.0000x reference)
import functools

import jax
import jax.numpy as jnp
from jax import lax
from jax.experimental import pallas as pl
from jax.experimental.pallas import tpu as pltpu
from jax.experimental.pallas import tpu_sc as plsc

S = 7
B = 2
NC = 20
IMG = 448.0
GRID = 64.0
COORD = 5.0
NOOBJ = 0.5
N = 128
NGT = 30
NCELL = S * S
L = 16
NTILES = 16
NBQ = 4
GPT = 2
CPT = 14


def _fsqrt(x):
    i = lax.bitcast_convert_type(x, jnp.int32)
    y = lax.bitcast_convert_type((i >> 1) + 0x1FBD1DF5, jnp.float32)
    y = 0.5 * (y + x / y)
    y = 0.5 * (y + x / y)
    return y


def _clip01(v):
    return jnp.minimum(jnp.maximum(v, 0.0), IMG - 1.0)


def _body(conf_h, box_h, cls_h, gt_h, lab_h, out_h,
          conf_v, box_v, cls_v, gt_v, lab_v,
          obj_s, ox_s, oy_s, ow_s, oh_s, lm_s,
          part_v, red_v, out_v, shared, sem):
    wid = lax.axis_index("s") + lax.axis_index("c")
    bq = wid % NBQ
    rg = wid // NBQ
    ilo = jnp.minimum(2 * rg, S - 2)
    vlo = CPT * rg
    vhi = jnp.minimum(vlo + CPT - 1, NCELL - 1)
    co7 = ilo * S
    cp_first = [pltpu.async_copy(lab_h, lab_v, sem),
                pltpu.async_copy(gt_h, gt_v, sem)]
    cp_rest = [pltpu.async_copy(box_h.at[pl.ds(ilo * (S * 8), 2 * S * 8)], box_v, sem)]
    for r in range(2):
        for cx in range(S):
            cp_rest.append(pltpu.async_copy(
                conf_h.at[ilo + r, cx], conf_v.at[pl.ds((r * S + cx) * B, B)], sem))
        for cc in range(NC):
            cp_rest.append(pltpu.async_copy(
                cls_h.at[ilo + r, cc], cls_v.at[pl.ds((r * NC + cc) * S, S)], sem))

    iota = lax.iota(jnp.int32, L)
    zf = jnp.zeros((L,), jnp.float32)
    zi = jnp.zeros((L,), jnp.int32)
    ones = jnp.ones((L,), jnp.float32)

    for k in range(CPT * GPT):
        sl = pl.ds(k * L, L)
        obj_s[sl] = zf
        ox_s[sl] = zf
        oy_s[sl] = zf
        ow_s[sl] = zf
        oh_s[sl] = zf
        lm_s[sl] = zi

    for cp in cp_first:
        cp.wait()

    def ebody(it, carry):
        grp = it // NGT
        j = it - grp * NGT
        n0 = pl.multiple_of(bq * (GPT * L) + grp * L, L)
        nsl = pl.ds(n0, L)
        x1 = gt_v[j * 4 + 0, nsl]
        y1 = gt_v[j * 4 + 1, nsl]
        x2 = gt_v[j * 4 + 2, nsl]
        y2 = gt_v[j * 4 + 3, nsl]
        lb = lab_v[j, nsl]
        w = x2 - x1
        h = y2 - y1
        x = x1 + w * 0.5
        y = y1 + h * 0.5
        cx = (x / GRID).astype(jnp.int32)
        cy = (y / GRID).astype(jnp.int32)
        cell = cy * S + cx
        m = (cell >= vlo) & (cell <= vhi)
        idx = (jnp.clip(cell - co7, 0, CPT - 1) * GPT + grp) * L + iota
        plsc.store_scatter(obj_s, [idx], ones, mask=m)
        plsc.store_scatter(ox_s, [idx], (x - cx.astype(jnp.float32) * GRID) / GRID, mask=m)
        plsc.store_scatter(oy_s, [idx], (y - cy.astype(jnp.float32) * GRID) / GRID, mask=m)
        plsc.store_scatter(ow_s, [idx], w / IMG, mask=m)
        plsc.store_scatter(oh_s, [idx], h / IMG, mask=m)
        cur = plsc.load_gather(lm_s, [idx], mask=m)
        plsc.store_scatter(lm_s, [idx], cur | jnp.left_shift(jnp.int32(1), lb), mask=m)
        return carry

    lax.fori_loop(0, GPT * NGT, ebody, 0, unroll=2)

    for cp in cp_rest:
        cp.wait()

    def cbody(it, acc):
        ci = it // GPT
        grp = it - ci * GPT
        n0 = pl.multiple_of(bq * (GPT * L) + grp * L, L)
        nsl = pl.ds(n0, L)
        lr = lax.div(ci, S)
        cxi = ci - lr * S
        cg = co7 + ci
        validf = jnp.where((cg >= vlo) & (cg <= vhi), 1.0, 0.0)
        ltx = cxi.astype(jnp.float32) * GRID
        lty = (ilo + lr).astype(jnp.float32) * GRID
        sl = pl.ds((ci * GPT + grp) * L, L)
        obj = obj_s[sl]
        gx = ox_s[sl]
        gy = oy_s[sl]
        gw = ow_s[sl]
        gh = oh_s[sl]
        lm = lm_s[sl]
        gcx = gx * GRID + ltx
        gcy = gy * GRID + lty
        gww = gw * IMG
        ghh = gh * IMG
        gx1 = _clip01(gcx - gww * 0.5)
        gy1 = _clip01(gcy - ghh * 0.5)
        gx2 = _clip01(gcx + gww * 0.5)
        gy2 = _clip01(gcy + ghh * 0.5)
        garea = (gx2 - gx1) * (gy2 - gy1)
        es, cfs, ious = [], [], []
        for b in range(B):
            ex = box_v[ci * 8 + b * 4 + 0, nsl]
            ey = box_v[ci * 8 + b * 4 + 1, nsl]
            ew = box_v[ci * 8 + b * 4 + 2, nsl]
            eh = box_v[ci * 8 + b * 4 + 3, nsl]
            cf = conf_v[ci * B + b, nsl]
            pcx = ex * GRID + ltx
            pcy = ey * GRID + lty
            pww = ew * IMG
            phh = eh * IMG
            px1 = _clip01(pcx - pww * 0.5)
            py1 = _clip01(pcy - phh * 0.5)
            px2 = _clip01(pcx + pww * 0.5)
            py2 = _clip01(pcy + phh * 0.5)
            parea = (px2 - px1) * (py2 - py1)
            ix = jnp.maximum(jnp.minimum(px2, gx2) - jnp.maximum(px1, gx1), 0.0)
            iy = jnp.maximum(jnp.minimum(py2, gy2) - jnp.maximum(py1, gy1), 0.0)
            inter = ix * iy
            ious.append(inter / (parea + garea - inter + 1e-4))
            es.append((ex, ey, ew, eh))
            cfs.append(cf)
        sel = jnp.where(ious[1] > ious[0], 1.0, 0.0)
        ioumax = jnp.maximum(ious[0], ious[1])
        bom = (obj * (1.0 - sel), obj * sel)
        sgw = _fsqrt(jnp.maximum(gw, 1e-8))
        sgh = _fsqrt(jnp.maximum(gh, 1e-8))
        contrib = zf
        sq = lambda v: v * v
        for b in range(B):
            ex, ey, ew, eh = es[b]
            cf = cfs[b]
            xy = sq(ex - gx) + sq(ey - gy)
            wwh = (sq(_fsqrt(jnp.maximum(ew, 1e-8)) - sgw)
                   + sq(_fsqrt(jnp.maximum(eh, 1e-8)) - sgh))
            co = sq(cf - ioumax)
            contrib = (contrib + bom[b] * (COORD * (xy + wwh) + co)
                       + NOOBJ * (1.0 - bom[b]) * cf * cf)
        ca = zf
        crow = lr * (NC * S) + cxi
        for cc in range(NC):
            clsv = cls_v[crow + cc * S, nsl]
            bit = (jnp.right_shift(lm, cc) & 1).astype(jnp.float32)
            d = clsv - bit
            ca = ca + d * d
        contrib = contrib + obj * ca
        return acc + contrib * validf

    acc = lax.fori_loop(0, CPT * GPT, cbody, zf)
    part_v[...] = acc
    pltpu.sync_copy(part_v, shared.at[pl.ds(wid * L, L)])
    plsc.subcore_barrier()

    @pl.when(wid == 0)
    def _():
        pltpu.sync_copy(shared, red_v)
        tot = zf
        for t in range(NTILES):
            tot = tot + red_v[pl.ds(t * L, L)]
        total = jnp.sum(tot) * jnp.float32(1.0 / N)
        out_v[...] = jnp.where(iota == 0, total, 0.0)
        pltpu.sync_copy(out_v, out_h)


_mesh = plsc.VectorSubcoreMesh(core_axis_name="c", subcore_axis_name="s",
                               num_cores=1, num_subcores=NTILES)

_sc_loss = functools.partial(
    pl.kernel,
    out_type=jax.ShapeDtypeStruct((L,), jnp.float32),
    mesh=_mesh,
    compiler_params=pltpu.CompilerParams(needs_layout_passes=False),
    scratch_types=[
        pltpu.VMEM((2 * S * B, 128), jnp.float32),
        pltpu.VMEM((2 * S * 8, 128), jnp.float32),
        pltpu.VMEM((2 * NC * S, 128), jnp.float32),
        pltpu.VMEM((NGT * 4, 128), jnp.float32),
        pltpu.VMEM((NGT, 128), jnp.int32),
        pltpu.VMEM((CPT * GPT * L,), jnp.float32),
        pltpu.VMEM((CPT * GPT * L,), jnp.float32),
        pltpu.VMEM((CPT * GPT * L,), jnp.float32),
        pltpu.VMEM((CPT * GPT * L,), jnp.float32),
        pltpu.VMEM((CPT * GPT * L,), jnp.float32),
        pltpu.VMEM((CPT * GPT * L,), jnp.int32),
        pltpu.VMEM((L,), jnp.float32),
        pltpu.VMEM((NTILES * L,), jnp.float32),
        pltpu.VMEM((L,), jnp.float32),
        pltpu.VMEM_SHARED((NTILES * L,), jnp.float32),
        pltpu.SemaphoreType.DMA,
    ],
)(_body)


def kernel(preConfidence, preBoxes, preCondClasses, groundTruth, groundLabels):
    confT = jnp.transpose(preConfidence, (1, 2, 3, 0))
    boxT = jnp.transpose(preBoxes, (1, 2, 3, 0)).reshape(S * S * B * 4, N)
    clsT = jnp.transpose(preCondClasses, (1, 3, 2, 0))
    gtT = jnp.transpose(groundTruth, (1, 2, 0)).reshape(NGT * 4, N)
    labT = jnp.transpose(groundLabels.astype(jnp.int32), (1, 0))
    out = _sc_loss(confT, boxT, clsT, gtT, labT)
    return out[0]


# speedup vs baseline: 1.0107x; 1.0107x over previous
import functools

import jax
import jax.numpy as jnp
from jax import lax
from jax.experimental import pallas as pl
from jax.experimental.pallas import tpu as pltpu
from jax.experimental.pallas import tpu_sc as plsc

S = 7
B = 2
NC = 20
IMG = 448.0
GRID = 64.0
COORD = 5.0
NOOBJ = 0.5
N = 128
NGT = 30
NCELL = S * S
L = 16
NTILES = 16
NBQ = 4
GPT = 2
CPT = 14


def _fsqrt(x):
    i = lax.bitcast_convert_type(x, jnp.int32)
    y = lax.bitcast_convert_type((i >> 1) + 0x1FBD1DF5, jnp.float32)
    y = 0.5 * (y + x / y)
    y = 0.5 * (y + x / y)
    return y


def _clip01(v):
    return jnp.minimum(jnp.maximum(v, 0.0), IMG - 1.0)


def _body(conf_h, box_h, cls_h, gt_h, lab_h, out_h,
          conf_v, box_v, cls_v, gt_v, lab_v,
          obj_s, ox_s, oy_s, ow_s, oh_s, lm_s,
          part_v, red_v, out_v, shared, sem):
    wid = lax.axis_index("s") + lax.axis_index("c")
    bq = wid % NBQ
    rg = wid // NBQ
    ilo = jnp.minimum(2 * rg, S - 2)
    vlo = CPT * rg
    vhi = jnp.minimum(vlo + CPT - 1, NCELL - 1)
    co7 = ilo * S
    cp_first = [pltpu.async_copy(lab_h, lab_v, sem),
                pltpu.async_copy(gt_h, gt_v, sem)]
    cp_rest = [pltpu.async_copy(box_h.at[pl.ds(ilo * (S * 8), 2 * S * 8)], box_v, sem)]
    for r in range(2):
        for cx in range(S):
            cp_rest.append(pltpu.async_copy(
                conf_h.at[ilo + r, cx], conf_v.at[pl.ds((r * S + cx) * B, B)], sem))
        for cc in range(NC):
            cp_rest.append(pltpu.async_copy(
                cls_h.at[ilo + r, cc], cls_v.at[pl.ds((r * NC + cc) * S, S)], sem))

    iota = lax.iota(jnp.int32, L)
    zf = jnp.zeros((L,), jnp.float32)
    zi = jnp.zeros((L,), jnp.int32)
    ones = jnp.ones((L,), jnp.float32)

    for k in range(CPT * GPT):
        sl = pl.ds(k * L, L)
        obj_s[sl] = zf
        ox_s[sl] = zf
        oy_s[sl] = zf
        ow_s[sl] = zf
        oh_s[sl] = zf
        lm_s[sl] = zi

    for cp in cp_first:
        cp.wait()

    def ebody(it, carry):
        grp = it // NGT
        j = it - grp * NGT
        n0 = pl.multiple_of(bq * (GPT * L) + grp * L, L)
        nsl = pl.ds(n0, L)
        x1 = gt_v[j * 4 + 0, nsl]
        y1 = gt_v[j * 4 + 1, nsl]
        x2 = gt_v[j * 4 + 2, nsl]
        y2 = gt_v[j * 4 + 3, nsl]
        lb = lab_v[j, nsl]
        w = x2 - x1
        h = y2 - y1
        x = x1 + w * 0.5
        y = y1 + h * 0.5
        cx = (x / GRID).astype(jnp.int32)
        cy = (y / GRID).astype(jnp.int32)
        cell = cy * S + cx
        m = (cell >= vlo) & (cell <= vhi)
        idx = (jnp.clip(cell - co7, 0, CPT - 1) * GPT + grp) * L + iota
        plsc.store_scatter(obj_s, [idx], ones, mask=m)
        plsc.store_scatter(ox_s, [idx], (x - cx.astype(jnp.float32) * GRID) / GRID, mask=m)
        plsc.store_scatter(oy_s, [idx], (y - cy.astype(jnp.float32) * GRID) / GRID, mask=m)
        plsc.store_scatter(ow_s, [idx], w / IMG, mask=m)
        plsc.store_scatter(oh_s, [idx], h / IMG, mask=m)
        cur = plsc.load_gather(lm_s, [idx], mask=m)
        plsc.store_scatter(lm_s, [idx], cur | jnp.left_shift(jnp.int32(1), lb), mask=m)
        return carry

    lax.fori_loop(0, GPT * NGT, ebody, 0)

    for cp in cp_rest:
        cp.wait()

    def cbody(it, acc):
        ci = it // GPT
        grp = it - ci * GPT
        n0 = pl.multiple_of(bq * (GPT * L) + grp * L, L)
        nsl = pl.ds(n0, L)
        lr = lax.div(ci, S)
        cxi = ci - lr * S
        cg = co7 + ci
        validf = jnp.where((cg >= vlo) & (cg <= vhi), 1.0, 0.0)
        ltx = cxi.astype(jnp.float32) * GRID
        lty = (ilo + lr).astype(jnp.float32) * GRID
        sl = pl.ds((ci * GPT + grp) * L, L)
        obj = obj_s[sl]
        gx = ox_s[sl]
        gy = oy_s[sl]
        gw = ow_s[sl]
        gh = oh_s[sl]
        lm = lm_s[sl]
        gcx = gx * GRID + ltx
        gcy = gy * GRID + lty
        gww = gw * IMG
        ghh = gh * IMG
        gx1 = _clip01(gcx - gww * 0.5)
        gy1 = _clip01(gcy - ghh * 0.5)
        gx2 = _clip01(gcx + gww * 0.5)
        gy2 = _clip01(gcy + ghh * 0.5)
        garea = (gx2 - gx1) * (gy2 - gy1)
        es, cfs, ious = [], [], []
        for b in range(B):
            ex = box_v[ci * 8 + b * 4 + 0, nsl]
            ey = box_v[ci * 8 + b * 4 + 1, nsl]
            ew = box_v[ci * 8 + b * 4 + 2, nsl]
            eh = box_v[ci * 8 + b * 4 + 3, nsl]
            cf = conf_v[ci * B + b, nsl]
            pcx = ex * GRID + ltx
            pcy = ey * GRID + lty
            pww = ew * IMG
            phh = eh * IMG
            px1 = _clip01(pcx - pww * 0.5)
            py1 = _clip01(pcy - phh * 0.5)
            px2 = _clip01(pcx + pww * 0.5)
            py2 = _clip01(pcy + phh * 0.5)
            parea = (px2 - px1) * (py2 - py1)
            ix = jnp.maximum(jnp.minimum(px2, gx2) - jnp.maximum(px1, gx1), 0.0)
            iy = jnp.maximum(jnp.minimum(py2, gy2) - jnp.maximum(py1, gy1), 0.0)
            inter = ix * iy
            ious.append(inter / (parea + garea - inter + 1e-4))
            es.append((ex, ey, ew, eh))
            cfs.append(cf)
        sel = jnp.where(ious[1] > ious[0], 1.0, 0.0)
        ioumax = jnp.maximum(ious[0], ious[1])
        bom = (obj * (1.0 - sel), obj * sel)
        sgw = _fsqrt(jnp.maximum(gw, 1e-8))
        sgh = _fsqrt(jnp.maximum(gh, 1e-8))
        contrib = zf
        sq = lambda v: v * v
        for b in range(B):
            ex, ey, ew, eh = es[b]
            cf = cfs[b]
            xy = sq(ex - gx) + sq(ey - gy)
            wwh = (sq(_fsqrt(jnp.maximum(ew, 1e-8)) - sgw)
                   + sq(_fsqrt(jnp.maximum(eh, 1e-8)) - sgh))
            co = sq(cf - ioumax)
            contrib = (contrib + bom[b] * (COORD * (xy + wwh) + co)
                       + NOOBJ * (1.0 - bom[b]) * cf * cf)
        ca = zf
        crow = lr * (NC * S) + cxi
        for cc in range(NC):
            clsv = cls_v[crow + cc * S, nsl]
            bit = (jnp.right_shift(lm, cc) & 1).astype(jnp.float32)
            d = clsv - bit
            ca = ca + d * d
        contrib = contrib + obj * ca
        return acc + contrib * validf

    acc = lax.fori_loop(0, CPT * GPT, cbody, zf)
    part_v[...] = acc
    pltpu.sync_copy(part_v, shared.at[pl.ds(wid * L, L)])
    plsc.subcore_barrier()

    @pl.when(wid == 0)
    def _():
        pltpu.sync_copy(shared, red_v)
        tot = zf
        for t in range(NTILES):
            tot = tot + red_v[pl.ds(t * L, L)]
        total = jnp.sum(tot) * jnp.float32(1.0 / N)
        out_v[...] = jnp.where(iota == 0, total, 0.0)
        pltpu.sync_copy(out_v, out_h)


_mesh = plsc.VectorSubcoreMesh(core_axis_name="c", subcore_axis_name="s",
                               num_cores=1, num_subcores=NTILES)

_sc_loss = functools.partial(
    pl.kernel,
    out_type=jax.ShapeDtypeStruct((L,), jnp.float32),
    mesh=_mesh,
    compiler_params=pltpu.CompilerParams(needs_layout_passes=False),
    scratch_types=[
        pltpu.VMEM((2 * S * B, 128), jnp.float32),
        pltpu.VMEM((2 * S * 8, 128), jnp.float32),
        pltpu.VMEM((2 * NC * S, 128), jnp.float32),
        pltpu.VMEM((NGT * 4, 128), jnp.float32),
        pltpu.VMEM((NGT, 128), jnp.int32),
        pltpu.VMEM((CPT * GPT * L,), jnp.float32),
        pltpu.VMEM((CPT * GPT * L,), jnp.float32),
        pltpu.VMEM((CPT * GPT * L,), jnp.float32),
        pltpu.VMEM((CPT * GPT * L,), jnp.float32),
        pltpu.VMEM((CPT * GPT * L,), jnp.float32),
        pltpu.VMEM((CPT * GPT * L,), jnp.int32),
        pltpu.VMEM((L,), jnp.float32),
        pltpu.VMEM((NTILES * L,), jnp.float32),
        pltpu.VMEM((L,), jnp.float32),
        pltpu.VMEM_SHARED((NTILES * L,), jnp.float32),
        pltpu.SemaphoreType.DMA,
    ],
)(_body)


def kernel(preConfidence, preBoxes, preCondClasses, groundTruth, groundLabels):
    confT = jnp.transpose(preConfidence, (1, 2, 3, 0))
    boxT = jnp.transpose(preBoxes, (1, 2, 3, 0)).reshape(S * S * B * 4, N)
    clsT = jnp.transpose(preCondClasses, (1, 3, 2, 0))
    gtT = jnp.transpose(groundTruth, (1, 2, 0)).reshape(NGT * 4, N)
    labT = jnp.transpose(groundLabels.astype(jnp.int32), (1, 0))
    out = _sc_loss(confT, boxT, clsT, gtT, labT)
    return out[0]
